# baseline (speedup 1.0000x reference)
import functools

import jax
import jax.numpy as jnp
from jax import lax
from jax.experimental import pallas as pl
from jax.experimental.pallas import tpu as pltpu
from jax.experimental.pallas import tpu_sc as plsc

NUM_TYPES = 4
VOCAB = 100000
EMBED = 32
N = 425984

NC = 2
NS = 16
NW = NC * NS
B_PER_W = N // NW
CHUNK = 1024
N_CHUNKS = B_PER_W // CHUNK

_mesh = plsc.VectorSubcoreMesh(core_axis_name="c", subcore_axis_name="s")


@functools.partial(
    pl.kernel,
    mesh=_mesh,
    out_type=jax.ShapeDtypeStruct((N, EMBED), jnp.float32),
    compiler_params=pltpu.CompilerParams(use_tc_tiling_on_sc=False),
    scratch_types=[
        pltpu.VMEM((CHUNK,), jnp.int32),
        pltpu.VMEM((CHUNK,), jnp.int32),
        pltpu.VMEM((CHUNK,), jnp.int32),
        pltpu.VMEM((CHUNK,), jnp.int32),
        pltpu.VMEM((CHUNK,), jnp.int32),
        pltpu.VMEM((CHUNK,), jnp.int32),
        pltpu.VMEM((CHUNK, EMBED), jnp.float32),
        pltpu.SemaphoreType.DMA,
    ],
)
def _hetero_gather(x_hbm, types_hbm, t0_hbm, t1_hbm, t2_hbm, t3_hbm, out_hbm,
                   x_v, t_v, i0_v, i1_v, i2_v, i3_v, rows_v, sem):
    wid = lax.axis_index("s") * NC + lax.axis_index("c")
    base_w = wid * B_PER_W

    def chunk_body(c, carry):
        base = base_w + c * CHUNK
        pltpu.sync_copy(x_hbm.at[pl.ds(base, CHUNK)], x_v)
        pltpu.sync_copy(types_hbm.at[pl.ds(base, CHUNK)], t_v)

        def idx_body(i, carry2):
            off = i * 16
            xv = x_v[pl.ds(off, 16)]
            tv = t_v[pl.ds(off, 16)]
            zero = jnp.zeros((16,), jnp.int32)
            i0_v[pl.ds(off, 16)] = jnp.where(tv == 0, xv, zero)
            i1_v[pl.ds(off, 16)] = jnp.where(tv == 1, xv, zero)
            i2_v[pl.ds(off, 16)] = jnp.where(tv == 2, xv, zero)
            i3_v[pl.ds(off, 16)] = jnp.where(tv == 3, xv, zero)
            return carry2

        lax.fori_loop(0, CHUNK // 16, idx_body, 0)
        pltpu.async_copy(t0_hbm.at[i0_v], rows_v, sem).wait()
        pltpu.async_copy(t1_hbm.at[i1_v], rows_v, sem, add=True).wait()
        pltpu.async_copy(t2_hbm.at[i2_v], rows_v, sem, add=True).wait()
        pltpu.async_copy(t3_hbm.at[i3_v], rows_v, sem, add=True).wait()
        pltpu.sync_copy(rows_v, out_hbm.at[pl.ds(base, CHUNK)])
        return carry

    lax.fori_loop(0, N_CHUNKS, chunk_body, 0)


def kernel(x, types, table_0, table_1, table_2, table_3):
    return _hetero_gather(x.astype(jnp.int32), types.astype(jnp.int32),
                          table_0, table_1, table_2, table_3)


# speedup vs baseline: 7.5155x; 7.5155x over previous
import functools

import jax
import jax.numpy as jnp
from jax import lax
from jax.experimental import pallas as pl
from jax.experimental.pallas import tpu as pltpu
from jax.experimental.pallas import tpu_sc as plsc

NUM_TYPES = 4
VOCAB = 100000
EMBED = 32
N = 425984

NC = 2
NS = 16
NW = NC * NS
B_PER_W = N // NW
CHUNK = 256
N_CHUNKS = B_PER_W // CHUNK

_mesh = plsc.VectorSubcoreMesh(core_axis_name="c", subcore_axis_name="s")


@functools.partial(
    pl.kernel,
    mesh=_mesh,
    out_type=jax.ShapeDtypeStruct((N, EMBED), jnp.float32),
    compiler_params=pltpu.CompilerParams(use_tc_tiling_on_sc=True,
                                         needs_layout_passes=False),
    scratch_types=[
        pltpu.VMEM((CHUNK,), jnp.int32),
        pltpu.VMEM((CHUNK,), jnp.int32),
        pltpu.VMEM((CHUNK, 4 * EMBED), jnp.float32),
        pltpu.VMEM((CHUNK, EMBED), jnp.float32),
        pltpu.SemaphoreType.DMA,
    ],
)
def _hetero_gather(x_hbm, types_hbm, table_hbm, out_hbm,
                   x_v, t_v, rows_v, seg_v, sem):
    wid = lax.axis_index("s") * NC + lax.axis_index("c")
    base_w = wid * B_PER_W
    iota = lax.iota(jnp.int32, 16)

    def chunk_body(c, carry):
        base = base_w + c * CHUNK
        pltpu.sync_copy(x_hbm.at[pl.ds(base, CHUNK)], x_v)
        pltpu.sync_copy(types_hbm.at[pl.ds(base, CHUNK)], t_v)
        pltpu.async_copy(table_hbm.at[x_v], rows_v, sem).wait()

        def group_body(i, carry2):
            r0 = i * 16
            t16 = t_v[pl.ds(r0, 16)]
            for j in range(16):
                r = r0 + j
                rowv = jnp.full((16,), r, jnp.int32)
                c0 = t16[j] * EMBED + iota
                g0 = plsc.load_gather(rows_v, [rowv, c0])
                g1 = plsc.load_gather(rows_v, [rowv, c0 + 16])
                seg_v[r, pl.ds(0, 16)] = g0
                seg_v[r, pl.ds(16, 16)] = g1
            return carry2

        lax.fori_loop(0, CHUNK // 16, group_body, 0)
        pltpu.sync_copy(seg_v, out_hbm.at[pl.ds(base, CHUNK)])
        return carry

    lax.fori_loop(0, N_CHUNKS, chunk_body, 0)


def kernel(x, types, table_0, table_1, table_2, table_3):
    table = jnp.concatenate([table_0, table_1, table_2, table_3], axis=1)
    return _hetero_gather(x.astype(jnp.int32), types.astype(jnp.int32), table)


# speedup vs baseline: 9.8322x; 1.3083x over previous
import functools

import jax
import jax.numpy as jnp
from jax import lax
from jax.experimental import pallas as pl
from jax.experimental.pallas import tpu as pltpu
from jax.experimental.pallas import tpu_sc as plsc

NUM_TYPES = 4
VOCAB = 100000
EMBED = 32
N = 425984

NC = 2
NS = 16
NW = NC * NS
B_PER_W = N // NW
CHUNK = 128
N_CHUNKS = B_PER_W // CHUNK

_mesh = plsc.VectorSubcoreMesh(core_axis_name="c", subcore_axis_name="s")


@functools.partial(
    pl.kernel,
    mesh=_mesh,
    out_type=jax.ShapeDtypeStruct((N, EMBED), jnp.float32),
    compiler_params=pltpu.CompilerParams(use_tc_tiling_on_sc=True),
    scratch_types=[
        [pltpu.VMEM((CHUNK,), jnp.int32)] * 2,
        [pltpu.VMEM((CHUNK,), jnp.int32)] * 2,
        [pltpu.VMEM((CHUNK, 4 * EMBED), jnp.float32)] * 2,
        [pltpu.VMEM((CHUNK, EMBED), jnp.float32)] * 2,
        [pltpu.SemaphoreType.DMA] * 2,
        [pltpu.SemaphoreType.DMA] * 2,
        [pltpu.SemaphoreType.DMA] * 2,
    ],
)
def _hetero_gather(x_hbm, types_hbm, table_hbm, out_hbm,
                   x_v, t_v, rows_v, seg_v, sem_xt, sem_g, sem_o):
    wid = lax.axis_index("s") * NC + lax.axis_index("c")
    base_w = wid * B_PER_W

    def fire_xt(c, b):
        base = base_w + c * CHUNK
        pltpu.async_copy(x_hbm.at[pl.ds(base, CHUNK)], x_v[b], sem_xt[b])
        pltpu.async_copy(types_hbm.at[pl.ds(base, CHUNK)], t_v[b], sem_xt[b])

    def wait_xt(c, b):
        base = base_w + c * CHUNK
        pltpu.make_async_copy(x_hbm.at[pl.ds(base, CHUNK)], x_v[b], sem_xt[b]).wait()
        pltpu.make_async_copy(types_hbm.at[pl.ds(base, CHUNK)], t_v[b], sem_xt[b]).wait()

    fire_xt(0, 0)
    fire_xt(1, 1)
    wait_xt(0, 0)
    pltpu.async_copy(table_hbm.at[x_v[0]], rows_v[0], sem_g[0])

    def outer(g, carry):
        for b in range(2):
            c = g * 2 + b
            nb = 1 - b

            @pl.when(c + 1 < N_CHUNKS)
            def _():
                wait_xt(c + 1, nb)
                pltpu.async_copy(table_hbm.at[x_v[nb]], rows_v[nb], sem_g[nb])

            pltpu.make_async_copy(table_hbm.at[x_v[b]], rows_v[b], sem_g[b]).wait()

            @pl.when(c + 2 < N_CHUNKS)
            def _():
                fire_xt(c + 2, b)

            @pl.when(c >= 2)
            def _():
                base_p = base_w + (c - 2) * CHUNK
                pltpu.make_async_copy(seg_v[b], out_hbm.at[pl.ds(base_p, CHUNK)],
                                      sem_o[b]).wait()

            def group_body(i, carry2):
                r0 = i * 16
                t16 = t_v[b][pl.ds(r0, 16)]
                for j in range(16):
                    r = r0 + j
                    col = t16[j] * EMBED
                    seg_v[b][r, pl.ds(0, 16)] = rows_v[b][r, pl.ds(col, 16)]
                    seg_v[b][r, pl.ds(16, 16)] = rows_v[b][r, pl.ds(col + 16, 16)]
                return carry2

            lax.fori_loop(0, CHUNK // 16, group_body, 0)

            base = base_w + c * CHUNK
            pltpu.async_copy(seg_v[b], out_hbm.at[pl.ds(base, CHUNK)], sem_o[b])
        return carry

    lax.fori_loop(0, N_CHUNKS // 2, outer, 0)

    for b in range(2):
        c = N_CHUNKS - 2 + b
        base = base_w + c * CHUNK
        pltpu.make_async_copy(seg_v[b], out_hbm.at[pl.ds(base, CHUNK)],
                              sem_o[b]).wait()


def kernel(x, types, table_0, table_1, table_2, table_3):
    table = jnp.concatenate([table_0, table_1, table_2, table_3], axis=1)
    return _hetero_gather(x.astype(jnp.int32), types.astype(jnp.int32), table)
